# baseline (speedup 1.0000x reference)
import functools

import jax
import jax.numpy as jnp
from jax import lax
from jax.experimental import pallas as pl
from jax.experimental.pallas import tpu as pltpu
from jax.experimental.pallas import tpu_sc as plsc

N = 10000
E = 320000
D = 128
HID = 16
HEADS = 8
OUT = 64

NC = 2
NS = 16
NW = NC * NS
B = 128
STEPS = 84
T = B * STEPS
EP = T * NW
NR = 10240
RPW = NR // NS

_MESH = dict(core_axis_name="c", subcore_axis_name="s")


def _proj1_body(x_ref, w_ref, ps_ref, pd_ref, h1_ref, as_ref, ad_ref):
    h1 = jnp.dot(x_ref[...], w_ref[...], preferred_element_type=jnp.float32)
    h1_ref[...] = h1
    as_ref[...] = jnp.dot(h1, ps_ref[...], preferred_element_type=jnp.float32)
    ad_ref[...] = jnp.dot(h1, pd_ref[...], preferred_element_type=jnp.float32)


def _mid_body(p0_ref, p1_ref, b1_ref, w2_ref, ps_ref, pd_ref,
              h2_ref, as_ref, ad_ref):
    h = jnp.maximum(p0_ref[...] + p1_ref[...] + b1_ref[...], 0.0)
    h2 = jnp.dot(h, w2_ref[...], preferred_element_type=jnp.float32)
    h2_ref[...] = h2
    as_ref[...] = jnp.dot(h2, ps_ref[...], preferred_element_type=jnp.float32)
    ad_ref[...] = jnp.dot(h2, pd_ref[...], preferred_element_type=jnp.float32)


def _comb_body(s0_ref, s1_ref, s_ref):
    s_ref[...] = s0_ref[...] + s1_ref[...] + 1e-16


def _final_body(q0_ref, q1_ref, b2_ref, out_ref):
    o = q0_ref[...] + q1_ref[...] + b2_ref[...]
    m = jnp.max(o, axis=1, keepdims=True)
    s = jnp.sum(jnp.exp(o - m), axis=1, keepdims=True)
    out_ref[...] = o - m - jnp.log(s)


def _make_att(nmask):
    @functools.partial(
        pl.kernel,
        out_type=[
            jax.ShapeDtypeStruct((EP, 16), jnp.float32),
            jax.ShapeDtypeStruct((NR, 16), jnp.float32),
            jax.ShapeDtypeStruct((NR, 16), jnp.float32),
        ],
        mesh=plsc.VectorSubcoreMesh(**_MESH),
        compiler_params=pltpu.CompilerParams(use_tc_tiling_on_sc=False),
        scratch_types=[
            pltpu.VMEM((6, 2, B), jnp.int32),
            pltpu.VMEM((3, B, 16), jnp.float32),
            pltpu.VMEM((3, B, 16), jnp.float32),
            pltpu.VMEM((3, B, 16), jnp.float32),
        ] + [pltpu.SemaphoreType.DMA] * 15 + [
            pltpu.VMEM_SHARED((NR, 16), jnp.float32),
        ],
    )
    def att(asp, adp, pih, ee_out, s0_out, s1_out,
            idxb, asg, adg, eev,
            si0, si1, si2, si3, si4, si5,
            sg0, sg1, sg2, ss0, ss1, ss2, sl0, sl1, sl2, s_acc):
        sis = (si0, si1, si2, si3, si4, si5)
        sgs = (sg0, sg1, sg2)
        sss = (ss0, ss1, ss2)
        sls = (sl0, sl1, sl2)
        cid = lax.axis_index("c")
        sid = lax.axis_index("s")
        wid = cid * NS + sid
        lanes = lax.iota(jnp.int32, 16)
        maskvec = jnp.where(lanes < nmask, 1.0, 0.0).astype(jnp.float32)
        zero16 = maskvec * 0.0

        def zrow(i, carry):
            eev[0, i, :] = zero16
            return carry
        lax.fori_loop(0, B, zrow, 0)
        for k in range(RPW // B):
            pltpu.sync_copy(eev.at[0], s_acc.at[pl.ds(sid * RPW + k * B, B)])
        plsc.subcore_barrier()

        def fire_idx(j, q):
            pltpu.async_copy(pih.at[wid].at[j], idxb.at[q], sis[q])

        def drain_idx(q):
            pltpu.make_async_copy(pih.at[0].at[0], idxb.at[q], sis[q]).wait()

        def fire_gathers(q, b):
            pltpu.async_copy(asp.at[idxb.at[q, 0]], asg.at[b], sgs[b])
            pltpu.async_copy(adp.at[idxb.at[q, 1]], adg.at[b], sgs[b])

        def drain_gathers(q, b):
            pltpu.make_async_copy(asp.at[idxb.at[q, 0]], asg.at[b],
                                  sgs[b]).wait()
            pltpu.make_async_copy(adp.at[idxb.at[q, 1]], adg.at[b],
                                  sgs[b]).wait()

        def fire_writes(j, q, b):
            off = pl.multiple_of(wid * T + j * B, B)
            pltpu.async_copy(eev.at[b], ee_out.at[pl.ds(off, B)], sls[b])
            pltpu.async_copy(eev.at[b], s_acc.at[idxb.at[q, 1]], sss[b],
                             add=True)

        def drain_writes(q, b):
            pltpu.make_async_copy(eev.at[b], ee_out.at[pl.ds(0, B)],
                                  sls[b]).wait()
            pltpu.make_async_copy(eev.at[b], s_acc.at[idxb.at[q, 1]],
                                  sss[b]).wait()

        def compute(b):
            def edge(i4, c2):
                for u in range(4):
                    i = i4 * 4 + u
                    v = asg[b, i, :] + adg[b, i, :]
                    v = jnp.maximum(v, 0.2 * v)
                    eev[b, i, :] = jnp.exp(v) * maskvec
                return c2
            lax.fori_loop(0, B // 4, edge, 0)

        for q in range(3):
            fire_idx(q, q)
        for q in range(2):
            drain_idx(q)
            fire_gathers(q, q)

        def body(jj, carry):
            for k in range(6):
                j = 6 * jj + k
                b = k % 3
                drain_gathers(k, b)
                compute(b)
                fire_writes(j, k, b)

                @pl.when(j >= 1)
                def _(qp=(k + 5) % 6, c=(k + 2) % 3):
                    drain_writes(qp, c)

                @pl.when(j + 2 < STEPS)
                def _(q2=(k + 2) % 6, c=(k + 2) % 3, j2=j + 2):
                    drain_idx(q2)
                    fire_gathers(q2, c)

                @pl.when(j + 3 < STEPS)
                def _(q3=(k + 3) % 6, j3=j + 3):
                    fire_idx(j3, q3)
            return carry
        lax.fori_loop(0, STEPS // 6, body, 0)
        drain_writes((STEPS - 1) % 6, (STEPS - 1) % 3)
        plsc.subcore_barrier()

        @pl.when(cid == 0)
        def _():
            pltpu.sync_copy(s_acc.at[pl.ds(sid * RPW, RPW)],
                            s0_out.at[pl.ds(sid * RPW, RPW)])

        @pl.when(cid == 1)
        def _():
            pltpu.sync_copy(s_acc.at[pl.ds(sid * RPW, RPW)],
                            s1_out.at[pl.ds(sid * RPW, RPW)])

    return att


def _make_msg(ch, per_head, bb, msteps):
    nch = ch // 16

    @functools.partial(
        pl.kernel,
        out_type=[
            jax.ShapeDtypeStruct((NR, ch), jnp.float32),
            jax.ShapeDtypeStruct((NR, ch), jnp.float32),
        ],
        mesh=plsc.VectorSubcoreMesh(**_MESH),
        compiler_params=pltpu.CompilerParams(use_tc_tiling_on_sc=False),
        scratch_types=[
            pltpu.VMEM((6, 2, bb), jnp.int32),
            pltpu.VMEM((3, bb, 16), jnp.float32),
            pltpu.VMEM((3, bb, 16), jnp.float32),
            pltpu.VMEM((3, bb, ch), jnp.float32),
        ] + [pltpu.SemaphoreType.DMA] * 15 + [
            pltpu.VMEM_SHARED((NR, ch), jnp.float32),
        ],
    )
    def msg(hh, pih, eeh, svh, o0_out, o1_out,
            idxb, eev, sv, hv,
            si0, si1, si2, si3, si4, si5,
            sg0, sg1, sg2, ss0, ss1, ss2, se0, se1, se2, out_acc):
        sis = (si0, si1, si2, si3, si4, si5)
        sgs = (sg0, sg1, sg2)
        sss = (ss0, ss1, ss2)
        ses = (se0, se1, se2)
        cid = lax.axis_index("c")
        sid = lax.axis_index("s")
        wid = cid * NS + sid
        lanes = lax.iota(jnp.int32, 16)
        zero16 = lanes.astype(jnp.float32) * 0.0

        def zrow(i, carry):
            for c in range(nch):
                hv[0, i, pl.ds(c * 16, 16)] = zero16
            return carry
        lax.fori_loop(0, bb, zrow, 0)
        for k in range(RPW // bb):
            pltpu.sync_copy(hv.at[0], out_acc.at[pl.ds(sid * RPW + k * bb, bb)])
        plsc.subcore_barrier()

        def fire_idx(j, q):
            pltpu.async_copy(pih.at[wid].at[j], idxb.at[q], sis[q])

        def drain_idx(q):
            pltpu.make_async_copy(pih.at[0].at[0], idxb.at[q], sis[q]).wait()

        def fire_gathers(j, q, b):
            off = pl.multiple_of(wid * T + j * bb, bb)
            pltpu.async_copy(hh.at[idxb.at[q, 0]], hv.at[b], sgs[b])
            pltpu.async_copy(eeh.at[pl.ds(off, bb)], eev.at[b], ses[b])
            pltpu.async_copy(svh.at[idxb.at[q, 1]], sv.at[b], sgs[b])

        def drain_gathers(q, b):
            pltpu.make_async_copy(hh.at[idxb.at[q, 0]], hv.at[b],
                                  sgs[b]).wait()
            pltpu.make_async_copy(eeh.at[pl.ds(0, bb)], eev.at[b],
                                  ses[b]).wait()
            pltpu.make_async_copy(svh.at[idxb.at[q, 1]], sv.at[b],
                                  sgs[b]).wait()

        def fire_scatter(q, b):
            pltpu.async_copy(hv.at[b], out_acc.at[idxb.at[q, 1]], sss[b],
                             add=True)

        def drain_scatter(q, b):
            pltpu.make_async_copy(hv.at[b], out_acc.at[idxb.at[q, 1]],
                                  sss[b]).wait()

        def compute(b):
            def edge(i2, c2):
                for u in range(2):
                    i = i2 * 2 + u
                    a = eev[b, i, :] / sv[b, i, :]
                    for c in range(nch):
                        lane = c if per_head else 0
                        sp = a.at[lanes * 0 + lane].get(
                            mode="promise_in_bounds")
                        hv[b, i, pl.ds(c * 16, 16)] = (
                            hv[b, i, pl.ds(c * 16, 16)] * sp)
                return c2
            lax.fori_loop(0, bb // 2, edge, 0)

        for q in range(3):
            fire_idx(q, q)
        for q in range(2):
            drain_idx(q)
            fire_gathers(q, q, q)

        def body(jj, carry):
            for k in range(6):
                j = 6 * jj + k
                b = k % 3
                drain_gathers(k, b)
                compute(b)
                fire_scatter(k, b)

                @pl.when(j >= 1)
                def _(qp=(k + 5) % 6, c=(k + 2) % 3):
                    drain_scatter(qp, c)

                @pl.when(j + 2 < msteps)
                def _(q2=(k + 2) % 6, c=(k + 2) % 3, j2=j + 2):
                    drain_idx(q2)
                    fire_gathers(j2, q2, c)

                @pl.when(j + 3 < msteps)
                def _(q3=(k + 3) % 6, j3=j + 3):
                    fire_idx(j3, q3)
            return carry
        lax.fori_loop(0, msteps // 6, body, 0)
        drain_scatter((msteps - 1) % 6, (msteps - 1) % 3)
        plsc.subcore_barrier()

        @pl.when(cid == 0)
        def _():
            pltpu.sync_copy(out_acc.at[pl.ds(sid * RPW, RPW)],
                            o0_out.at[pl.ds(sid * RPW, RPW)])

        @pl.when(cid == 1)
        def _():
            pltpu.sync_copy(out_acc.at[pl.ds(sid * RPW, RPW)],
                            o1_out.at[pl.ds(sid * RPW, RPW)])

    return msg


def kernel(x, edge_index, W1, a1s, a1d, b1, W2, a2s, a2d, b2):
    _att1 = _make_att(HEADS)
    _att2 = _make_att(1)
    _msg1 = _make_msg(HEADS * HID, True, 64, 168)
    _msg2 = _make_msg(OUT, False, B, STEPS)

    ei = edge_index.astype(jnp.int32)
    ar = jnp.arange(N, dtype=jnp.int32)
    pad = EP - (E + N)
    src = jnp.concatenate([ei[0], ar, jnp.zeros((pad,), jnp.int32)])
    dst = jnp.concatenate(
        [ei[1], ar, N + (jnp.arange(pad, dtype=jnp.int32) % (NR - N))])
    pih = jnp.stack([src.reshape(NW, STEPS, B), dst.reshape(NW, STEPS, B)],
                    axis=2)
    pih64 = jnp.stack([src.reshape(NW, 2 * STEPS, B // 2),
                       dst.reshape(NW, 2 * STEPS, B // 2)], axis=2)

    idx = jnp.arange(HEADS * HID, dtype=jnp.int32)
    P1s = jnp.zeros((HEADS * HID, 16), jnp.float32).at[idx, idx // HID].set(
        a1s.reshape(-1))
    P1d = jnp.zeros((HEADS * HID, 16), jnp.float32).at[idx, idx // HID].set(
        a1d.reshape(-1))
    P2s = jnp.zeros((OUT, 16), jnp.float32).at[:, 0].set(a2s[0])
    P2d = jnp.zeros((OUT, 16), jnp.float32).at[:, 0].set(a2d[0])

    h1, as1, ad1 = pl.pallas_call(
        _proj1_body,
        out_shape=[
            jax.ShapeDtypeStruct((N, HEADS * HID), jnp.float32),
            jax.ShapeDtypeStruct((N, 16), jnp.float32),
            jax.ShapeDtypeStruct((N, 16), jnp.float32),
        ],
    )(x, W1, P1s, P1d)
    ad1p = jnp.pad(ad1, ((0, NR - N), (0, 0)))

    ee1, s10, s11 = _att1(as1, ad1p, pih)
    sv1 = pl.pallas_call(
        _comb_body,
        out_shape=jax.ShapeDtypeStruct((NR, 16), jnp.float32),
    )(s10, s11)
    o10, o11 = _msg1(h1, pih64, ee1, sv1)

    h2, as2, ad2 = pl.pallas_call(
        _mid_body,
        out_shape=[
            jax.ShapeDtypeStruct((N, OUT), jnp.float32),
            jax.ShapeDtypeStruct((N, 16), jnp.float32),
            jax.ShapeDtypeStruct((N, 16), jnp.float32),
        ],
    )(o10[:N], o11[:N], b1.reshape(1, HEADS * HID), W2, P2s, P2d)
    ad2p = jnp.pad(ad2, ((0, NR - N), (0, 0)))

    ee2, s20, s21 = _att2(as2, ad2p, pih)
    sv2 = pl.pallas_call(
        _comb_body,
        out_shape=jax.ShapeDtypeStruct((NR, 16), jnp.float32),
    )(s20, s21)
    o20, o21 = _msg2(h2, pih, ee2, sv2)

    return pl.pallas_call(
        _final_body,
        out_shape=jax.ShapeDtypeStruct((N, OUT), jnp.float32),
    )(o20[:N], o21[:N], b2.reshape(1, OUT))


# speedup vs baseline: 1.0326x; 1.0326x over previous
import functools

import jax
import jax.numpy as jnp
from jax import lax
from jax.experimental import pallas as pl
from jax.experimental.pallas import tpu as pltpu
from jax.experimental.pallas import tpu_sc as plsc

N = 10000
E = 320000
D = 128
HID = 16
HEADS = 8
OUT = 64

NC = 2
NS = 16
NW = NC * NS
B = 128
STEPS = 84
T = B * STEPS
EP = T * NW
NR = 10240
RPW = NR // NS

_MESH = dict(core_axis_name="c", subcore_axis_name="s")


def _proj1_body(x_ref, w_ref, ps_ref, pd_ref, h1_ref, as_ref, ad_ref):
    h1 = jnp.dot(x_ref[...], w_ref[...], preferred_element_type=jnp.float32)
    h1_ref[...] = h1
    as_ref[...] = jnp.dot(h1, ps_ref[...], preferred_element_type=jnp.float32)
    ad_ref[...] = jnp.dot(h1, pd_ref[...], preferred_element_type=jnp.float32)


def _mid_body(p0_ref, p1_ref, b1_ref, w2_ref, ps_ref, pd_ref,
              h2_ref, as_ref, ad_ref):
    h = jnp.maximum(p0_ref[...] + p1_ref[...] + b1_ref[...], 0.0)
    h2 = jnp.dot(h, w2_ref[...], preferred_element_type=jnp.float32)
    h2_ref[...] = h2
    as_ref[...] = jnp.dot(h2, ps_ref[...], preferred_element_type=jnp.float32)
    ad_ref[...] = jnp.dot(h2, pd_ref[...], preferred_element_type=jnp.float32)


def _comb_body(s0_ref, s1_ref, s_ref):
    s_ref[...] = s0_ref[...] + s1_ref[...] + 1e-16


def _final_body(q0_ref, q1_ref, b2_ref, out_ref):
    o = q0_ref[...] + q1_ref[...] + b2_ref[...]
    m = jnp.max(o, axis=1, keepdims=True)
    s = jnp.sum(jnp.exp(o - m), axis=1, keepdims=True)
    out_ref[...] = o - m - jnp.log(s)


def _make_att(nmask, sa, sb):
    @functools.partial(
        pl.kernel,
        out_type=[
            jax.ShapeDtypeStruct((EP, 16), jnp.float32),
            jax.ShapeDtypeStruct((NR, 16), jnp.float32),
            jax.ShapeDtypeStruct((NR, 16), jnp.float32),
        ],
        mesh=plsc.VectorSubcoreMesh(**_MESH),
        compiler_params=pltpu.CompilerParams(use_tc_tiling_on_sc=False),
        scratch_types=[
            pltpu.VMEM((6, 2, B), jnp.int32),
            pltpu.VMEM((3, B, 16), jnp.float32),
            pltpu.VMEM((3, B, 16), jnp.float32),
            pltpu.VMEM((3, B, 16), jnp.float32),
        ] + [pltpu.SemaphoreType.DMA] * 15 + [
            pltpu.VMEM_SHARED((NR, 16), jnp.float32),
        ],
    )
    def att(asp, adp, pih, ee_out, s0_out, s1_out,
            idxb, asg, adg, eev,
            si0, si1, si2, si3, si4, si5,
            sg0, sg1, sg2, ss0, ss1, ss2, sl0, sl1, sl2, s_acc):
        sis = (si0, si1, si2, si3, si4, si5)
        sgs = (sg0, sg1, sg2)
        sss = (ss0, ss1, ss2)
        sls = (sl0, sl1, sl2)
        cid = lax.axis_index("c")
        sid = lax.axis_index("s")
        gbase = jnp.where(cid == 0, sid * sa, NS * sa + sid * sb)
        mysteps = jnp.where(cid == 0, sa, sb)
        iters = jnp.where(cid == 0, sa // 6, sb // 6)
        lanes = lax.iota(jnp.int32, 16)
        maskvec = jnp.where(lanes < nmask, 1.0, 0.0).astype(jnp.float32)
        zero16 = maskvec * 0.0

        def zrow(i, carry):
            eev[0, i, :] = zero16
            return carry
        lax.fori_loop(0, B, zrow, 0)
        for k in range(RPW // B):
            pltpu.sync_copy(eev.at[0], s_acc.at[pl.ds(sid * RPW + k * B, B)])
        plsc.subcore_barrier()

        def fire_idx(j, q):
            pltpu.async_copy(pih.at[gbase + j], idxb.at[q], sis[q])

        def drain_idx(q):
            pltpu.make_async_copy(pih.at[0], idxb.at[q], sis[q]).wait()

        def fire_gathers(q, b):
            pltpu.async_copy(asp.at[idxb.at[q, 0]], asg.at[b], sgs[b])
            pltpu.async_copy(adp.at[idxb.at[q, 1]], adg.at[b], sgs[b])

        def drain_gathers(q, b):
            pltpu.make_async_copy(asp.at[idxb.at[q, 0]], asg.at[b],
                                  sgs[b]).wait()
            pltpu.make_async_copy(adp.at[idxb.at[q, 1]], adg.at[b],
                                  sgs[b]).wait()

        def fire_writes(j, q, b):
            off = pl.multiple_of((gbase + j) * B, B)
            pltpu.async_copy(eev.at[b], ee_out.at[pl.ds(off, B)], sls[b])
            pltpu.async_copy(eev.at[b], s_acc.at[idxb.at[q, 1]], sss[b],
                             add=True)

        def drain_writes(q, b):
            pltpu.make_async_copy(eev.at[b], ee_out.at[pl.ds(0, B)],
                                  sls[b]).wait()
            pltpu.make_async_copy(eev.at[b], s_acc.at[idxb.at[q, 1]],
                                  sss[b]).wait()

        def compute(b):
            def edge(i4, c2):
                for u in range(4):
                    i = i4 * 4 + u
                    v = asg[b, i, :] + adg[b, i, :]
                    v = jnp.maximum(v, 0.2 * v)
                    eev[b, i, :] = jnp.exp(v) * maskvec
                return c2
            lax.fori_loop(0, B // 4, edge, 0)

        for q in range(3):
            fire_idx(q, q)
        for q in range(2):
            drain_idx(q)
            fire_gathers(q, q)

        def body(jj, carry):
            for k in range(6):
                j = 6 * jj + k
                b = k % 3
                drain_gathers(k, b)
                compute(b)
                fire_writes(j, k, b)

                @pl.when(j >= 1)
                def _(qp=(k + 5) % 6, c=(k + 2) % 3):
                    drain_writes(qp, c)

                @pl.when(j + 2 < mysteps)
                def _(q2=(k + 2) % 6, c=(k + 2) % 3, j2=j + 2):
                    drain_idx(q2)
                    fire_gathers(q2, c)

                @pl.when(j + 3 < mysteps)
                def _(q3=(k + 3) % 6, j3=j + 3):
                    fire_idx(j3, q3)
            return carry
        lax.fori_loop(0, iters, body, 0)
        drain_writes(5, 2)
        plsc.subcore_barrier()

        @pl.when(cid == 0)
        def _():
            pltpu.sync_copy(s_acc.at[pl.ds(sid * RPW, RPW)],
                            s0_out.at[pl.ds(sid * RPW, RPW)])

        @pl.when(cid == 1)
        def _():
            pltpu.sync_copy(s_acc.at[pl.ds(sid * RPW, RPW)],
                            s1_out.at[pl.ds(sid * RPW, RPW)])

    return att


def _make_msg(ch, per_head, bb, sa, sb):
    nch = ch // 16

    @functools.partial(
        pl.kernel,
        out_type=[
            jax.ShapeDtypeStruct((NR, ch), jnp.float32),
            jax.ShapeDtypeStruct((NR, ch), jnp.float32),
        ],
        mesh=plsc.VectorSubcoreMesh(**_MESH),
        compiler_params=pltpu.CompilerParams(use_tc_tiling_on_sc=False),
        scratch_types=[
            pltpu.VMEM((6, 2, bb), jnp.int32),
            pltpu.VMEM((3, bb, 16), jnp.float32),
            pltpu.VMEM((3, bb, 16), jnp.float32),
            pltpu.VMEM((3, bb, ch), jnp.float32),
        ] + [pltpu.SemaphoreType.DMA] * 15 + [
            pltpu.VMEM_SHARED((NR, ch), jnp.float32),
        ],
    )
    def msg(hh, pih, eeh, svh, o0_out, o1_out,
            idxb, eev, sv, hv,
            si0, si1, si2, si3, si4, si5,
            sg0, sg1, sg2, ss0, ss1, ss2, se0, se1, se2, out_acc):
        sis = (si0, si1, si2, si3, si4, si5)
        sgs = (sg0, sg1, sg2)
        sss = (ss0, ss1, ss2)
        ses = (se0, se1, se2)
        cid = lax.axis_index("c")
        sid = lax.axis_index("s")
        gbase = jnp.where(cid == 0, sid * sa, NS * sa + sid * sb)
        mysteps = jnp.where(cid == 0, sa, sb)
        iters = jnp.where(cid == 0, sa // 6, sb // 6)
        lanes = lax.iota(jnp.int32, 16)
        zero16 = lanes.astype(jnp.float32) * 0.0

        def zrow(i, carry):
            for c in range(nch):
                hv[0, i, pl.ds(c * 16, 16)] = zero16
            return carry
        lax.fori_loop(0, bb, zrow, 0)
        for k in range(RPW // bb):
            pltpu.sync_copy(hv.at[0], out_acc.at[pl.ds(sid * RPW + k * bb, bb)])
        plsc.subcore_barrier()

        def fire_idx(j, q):
            pltpu.async_copy(pih.at[gbase + j], idxb.at[q], sis[q])

        def drain_idx(q):
            pltpu.make_async_copy(pih.at[0], idxb.at[q], sis[q]).wait()

        def fire_gathers(j, q, b):
            off = pl.multiple_of((gbase + j) * bb, bb)
            pltpu.async_copy(hh.at[idxb.at[q, 0]], hv.at[b], sgs[b])
            pltpu.async_copy(eeh.at[pl.ds(off, bb)], eev.at[b], ses[b])
            pltpu.async_copy(svh.at[idxb.at[q, 1]], sv.at[b], sgs[b])

        def drain_gathers(q, b):
            pltpu.make_async_copy(hh.at[idxb.at[q, 0]], hv.at[b],
                                  sgs[b]).wait()
            pltpu.make_async_copy(eeh.at[pl.ds(0, bb)], eev.at[b],
                                  ses[b]).wait()
            pltpu.make_async_copy(svh.at[idxb.at[q, 1]], sv.at[b],
                                  sgs[b]).wait()

        def fire_scatter(q, b):
            pltpu.async_copy(hv.at[b], out_acc.at[idxb.at[q, 1]], sss[b],
                             add=True)

        def drain_scatter(q, b):
            pltpu.make_async_copy(hv.at[b], out_acc.at[idxb.at[q, 1]],
                                  sss[b]).wait()

        def compute(b):
            def edge(i2, c2):
                for u in range(2):
                    i = i2 * 2 + u
                    a = eev[b, i, :] / sv[b, i, :]
                    for c in range(nch):
                        lane = c if per_head else 0
                        sp = a.at[lanes * 0 + lane].get(
                            mode="promise_in_bounds")
                        hv[b, i, pl.ds(c * 16, 16)] = (
                            hv[b, i, pl.ds(c * 16, 16)] * sp)
                return c2
            lax.fori_loop(0, bb // 2, edge, 0)

        for q in range(3):
            fire_idx(q, q)
        for q in range(2):
            drain_idx(q)
            fire_gathers(q, q, q)

        def body(jj, carry):
            for k in range(6):
                j = 6 * jj + k
                b = k % 3
                drain_gathers(k, b)
                compute(b)
                fire_scatter(k, b)

                @pl.when(j >= 1)
                def _(qp=(k + 5) % 6, c=(k + 2) % 3):
                    drain_scatter(qp, c)

                @pl.when(j + 2 < mysteps)
                def _(q2=(k + 2) % 6, c=(k + 2) % 3, j2=j + 2):
                    drain_idx(q2)
                    fire_gathers(j2, q2, c)

                @pl.when(j + 3 < mysteps)
                def _(q3=(k + 3) % 6, j3=j + 3):
                    fire_idx(j3, q3)
            return carry
        lax.fori_loop(0, iters, body, 0)
        drain_scatter(5, 2)
        plsc.subcore_barrier()

        @pl.when(cid == 0)
        def _():
            pltpu.sync_copy(out_acc.at[pl.ds(sid * RPW, RPW)],
                            o0_out.at[pl.ds(sid * RPW, RPW)])

        @pl.when(cid == 1)
        def _():
            pltpu.sync_copy(out_acc.at[pl.ds(sid * RPW, RPW)],
                            o1_out.at[pl.ds(sid * RPW, RPW)])

    return msg


def kernel(x, edge_index, W1, a1s, a1d, b1, W2, a2s, a2d, b2):
    _att1 = _make_att(HEADS, 108, 60)
    _att2 = _make_att(1, 108, 60)
    _msg1 = _make_msg(HEADS * HID, True, 64, 252, 84)
    _msg2 = _make_msg(OUT, False, B, 114, 54)

    ei = edge_index.astype(jnp.int32)
    ar = jnp.arange(N, dtype=jnp.int32)
    pad = EP - (E + N)
    src = jnp.concatenate([ei[0], ar, jnp.zeros((pad,), jnp.int32)])
    dst = jnp.concatenate(
        [ei[1], ar, N + (jnp.arange(pad, dtype=jnp.int32) % (NR - N))])
    pih = jnp.stack([src.reshape(-1, B), dst.reshape(-1, B)], axis=1)
    pih64 = jnp.stack([src.reshape(-1, B // 2), dst.reshape(-1, B // 2)],
                      axis=1)

    idx = jnp.arange(HEADS * HID, dtype=jnp.int32)
    P1s = jnp.zeros((HEADS * HID, 16), jnp.float32).at[idx, idx // HID].set(
        a1s.reshape(-1))
    P1d = jnp.zeros((HEADS * HID, 16), jnp.float32).at[idx, idx // HID].set(
        a1d.reshape(-1))
    P2s = jnp.zeros((OUT, 16), jnp.float32).at[:, 0].set(a2s[0])
    P2d = jnp.zeros((OUT, 16), jnp.float32).at[:, 0].set(a2d[0])

    h1, as1, ad1 = pl.pallas_call(
        _proj1_body,
        out_shape=[
            jax.ShapeDtypeStruct((N, HEADS * HID), jnp.float32),
            jax.ShapeDtypeStruct((N, 16), jnp.float32),
            jax.ShapeDtypeStruct((N, 16), jnp.float32),
        ],
    )(x, W1, P1s, P1d)
    ad1p = jnp.pad(ad1, ((0, NR - N), (0, 0)))

    ee1, s10, s11 = _att1(as1, ad1p, pih)
    sv1 = pl.pallas_call(
        _comb_body,
        out_shape=jax.ShapeDtypeStruct((NR, 16), jnp.float32),
    )(s10, s11)
    o10, o11 = _msg1(h1, pih64, ee1, sv1)

    h2, as2, ad2 = pl.pallas_call(
        _mid_body,
        out_shape=[
            jax.ShapeDtypeStruct((N, OUT), jnp.float32),
            jax.ShapeDtypeStruct((N, 16), jnp.float32),
            jax.ShapeDtypeStruct((N, 16), jnp.float32),
        ],
    )(o10[:N], o11[:N], b1.reshape(1, HEADS * HID), W2, P2s, P2d)
    ad2p = jnp.pad(ad2, ((0, NR - N), (0, 0)))

    ee2, s20, s21 = _att2(as2, ad2p, pih)
    sv2 = pl.pallas_call(
        _comb_body,
        out_shape=jax.ShapeDtypeStruct((NR, 16), jnp.float32),
    )(s20, s21)
    o20, o21 = _msg2(h2, pih, ee2, sv2)

    return pl.pallas_call(
        _final_body,
        out_shape=jax.ShapeDtypeStruct((N, OUT), jnp.float32),
    )(o20[:N], o21[:N], b2.reshape(1, OUT))
